# baseline (speedup 1.0000x reference)
import jax
import jax.numpy as jnp
from jax.experimental import pallas as pl


def kernel(pred_disp, coords, prop_E, prop_A, prop_I22, connectivity):
    nA = connectivity[:, 0]
    nB = connectivity[:, 1]
    dx0 = coords[nB, 0] - coords[nA, 0]
    dz0 = coords[nB, 2] - coords[nA, 2]
    l0 = jnp.sqrt(dx0 ** 2 + dz0 ** 2)
    c = dx0 / l0
    s = dz0 / l0
    EA = prop_E * prop_A
    EI = prop_E * prop_I22
    dA = jnp.take(pred_disp, nA, axis=0)
    dB = jnp.take(pred_disp, nB, axis=0)
    d_global = jnp.concatenate([dA, dB], axis=1)
    ua = c * d_global[:, 0] + s * d_global[:, 1]
    wa = -s * d_global[:, 0] + c * d_global[:, 1]
    ta = d_global[:, 2]
    ub = c * d_global[:, 3] + s * d_global[:, 4]
    wb = -s * d_global[:, 3] + c * d_global[:, 4]
    tb = d_global[:, 5]
    d_local = jnp.stack([ua, wa, ta, ub, wb, tb], axis=1)
    L = l0
    L2 = l0 * l0
    L3 = L2 * l0
    f0 = EA / L * (ua - ub)
    f1 = 12 * EI / L3 * (wa - wb) + 6 * EI / L2 * (ta + tb)
    f2 = 6 * EI / L2 * (wa - wb) + EI / L * (4 * ta + 2 * tb)
    f3 = EA / L * (ub - ua)
    f4 = 12 * EI / L3 * (wb - wa) - 6 * EI / L2 * (ta + tb)
    f5 = 6 * EI / L2 * (wa - wb) + EI / L * (2 * ta + 4 * tb)
    f_local = jnp.stack([f0, f1, f2, f3, f4, f5], axis=1)
    g0 = c * f0 - s * f1
    g1 = s * f0 + c * f1
    g2 = f2
    g3 = c * f3 - s * f4
    g4 = s * f3 + c * f4
    g5 = f5
    f_global = jnp.stack([g0, g1, g2, g3, g4, g5], axis=1)
    F_global_A = f_global[:, 0:3]
    F_global_B = f_global[:, 3:6]
    N_count = pred_disp.shape[0]
    nodal_forces = jnp.zeros((N_count, 3), dtype=pred_disp.dtype)
    nodal_forces = nodal_forces.at[nA].add(F_global_A)
    nodal_forces = nodal_forces.at[nB].add(F_global_B)
    N_e = f3
    V_e = f4
    M1_e = f2
    M2_e = f5
    u_l = ub - ua
    theta1_l = ta
    theta2_l = tb
    phi = (wb - wa) / l0
    return (N_e, M1_e, M2_e, V_e, F_global_A, F_global_B, nodal_forces,
            f_local, f_global, d_local, u_l, theta1_l, theta2_l, phi, l0, c, s)


# speedup vs baseline: 12.5194x; 12.5194x over previous
import jax
import jax.numpy as jnp
from jax import lax
from jax.experimental import pallas as pl
from jax.experimental.pallas import tpu as pltpu
from jax.experimental.pallas import tpu_sc as plsc

NC = 2
NS = 16
NW = NC * NS
LANES = 16

C = 2000
SUB = 80
NSUB = C // SUB
VITERS = C // LANES

_MAGIC = 0x5F3759DF


def _rsqrt(x):
    xi = lax.bitcast_convert_type(x, jnp.int32)
    y = lax.bitcast_convert_type(
        jnp.int32(_MAGIC) - lax.shift_right_logical(xi, 1), jnp.float32)
    for _ in range(3):
        y = y * (1.5 - 0.5 * x * y * y)
    return y


def _beam_body(tdx, tdy, tth, tcx, tcz, na1, nb1, na3, nb3, p_e, p_a, p_i, z1,
               fl_o, fg_o, dl_o, ul_o, phi_o, l0_o, c_o, s_o,
               p0x_o, p0y_o, p0z_o, p1x_o, p1y_o, p1z_o,
               idxa_v, idxb_v, idxa2_v, idxb2_v, pe_v, pav_v, pi_v,
               dax_v, day_v, taa_v, cax_v, caz_v,
               dbx_v, dby_v, tbb_v, cbx_v, cbz_v,
               ul_v, phi_v, l0_v, cc_v, ss_v,
               fl_v, fg_v, dl_v,
               fax_v, fay_v, faz_v, fbx_v, fby_v, fbz_v,
               accx, accy, accz, sem):
    core = lax.axis_index("c")
    sub = lax.axis_index("s")
    wid = sub * NC + core
    n_edges = ul_o.shape[0]
    cpw = n_edges // (NW * C)

    @pl.when(sub == 0)
    def _zero():
        pltpu.sync_copy(z1, accx)
        pltpu.sync_copy(z1, accy)
        pltpu.sync_copy(z1, accz)

    plsc.subcore_barrier()

    lane = lax.iota(jnp.int32, LANES)
    lane6 = lane * 6

    def chunk_body(j, _):
        ck = wid * cpw + j
        base = ck * C
        pltpu.sync_copy(na1.at[pl.ds(base, C)], idxa_v)
        pltpu.sync_copy(nb1.at[pl.ds(base, C)], idxb_v)
        pltpu.sync_copy(na3.at[ck], idxa2_v)
        pltpu.sync_copy(nb3.at[ck], idxb2_v)
        pltpu.sync_copy(p_e.at[pl.ds(base, C)], pe_v)
        pltpu.sync_copy(p_a.at[pl.ds(base, C)], pav_v)
        pltpu.sync_copy(p_i.at[pl.ds(base, C)], pi_v)

        gathers = [
            pltpu.async_copy(tdx.at[idxa_v], dax_v, sem),
            pltpu.async_copy(tdy.at[idxa_v], day_v, sem),
            pltpu.async_copy(tth.at[idxa_v], taa_v, sem),
            pltpu.async_copy(tcx.at[idxa_v], cax_v, sem),
            pltpu.async_copy(tcz.at[idxa_v], caz_v, sem),
            pltpu.async_copy(tdx.at[idxb_v], dbx_v, sem),
            pltpu.async_copy(tdy.at[idxb_v], dby_v, sem),
            pltpu.async_copy(tth.at[idxb_v], tbb_v, sem),
            pltpu.async_copy(tcx.at[idxb_v], cbx_v, sem),
            pltpu.async_copy(tcz.at[idxb_v], cbz_v, sem),
        ]
        for g in gathers:
            g.wait()

        def vec(i, _):
            eb = i * LANES
            sl = pl.ds(eb, LANES)
            d_ax = dax_v[sl]
            d_ay = day_v[sl]
            ta = taa_v[sl]
            cax = cax_v[sl]
            caz = caz_v[sl]
            d_bx = dbx_v[sl]
            d_by = dby_v[sl]
            tb = tbb_v[sl]
            cbx = cbx_v[sl]
            cbz = cbz_v[sl]
            pe = pe_v[sl]
            pa = pav_v[sl]
            pi = pi_v[sl]

            dx0 = cbx - cax
            dz0 = cbz - caz
            x = dx0 * dx0 + dz0 * dz0
            rl = _rsqrt(x)
            l0 = x * rl
            cv = dx0 * rl
            sv = dz0 * rl
            ea = pe * pa
            ei = pe * pi
            ua = cv * d_ax + sv * d_ay
            wa = cv * d_ay - sv * d_ax
            ub = cv * d_bx + sv * d_by
            wb = cv * d_by - sv * d_bx
            rl2 = rl * rl
            rl3 = rl2 * rl
            f0 = ea * rl * (ua - ub)
            wab = wa - wb
            eil3 = ei * rl3
            eil2 = ei * rl2
            eil = ei * rl
            f1 = 12.0 * eil3 * wab + 6.0 * eil2 * (ta + tb)
            f2 = 6.0 * eil2 * wab + eil * (4.0 * ta + 2.0 * tb)
            f5 = 6.0 * eil2 * wab + eil * (2.0 * ta + 4.0 * tb)
            f3 = -f0
            f4 = -f1
            g0 = cv * f0 - sv * f1
            g1 = sv * f0 + cv * f1
            g3 = -g0
            g4 = -g1

            ul_v[sl] = ub - ua
            phi_v[sl] = (wb - wa) * rl
            l0_v[sl] = l0
            cc_v[sl] = cv
            ss_v[sl] = sv
            fax_v[sl] = g0
            fay_v[sl] = g1
            faz_v[sl] = f2
            fbx_v[sl] = g3
            fby_v[sl] = g4
            fbz_v[sl] = f5
            r6 = eb * 6 + lane6
            for col, val in ((0, f0), (1, f1), (2, f2), (3, f3), (4, f4),
                             (5, f5)):
                plsc.store_scatter(fl_v, [r6 + col], val)
            for col, val in ((0, g0), (1, g1), (2, f2), (3, g3), (4, g4),
                             (5, f5)):
                plsc.store_scatter(fg_v, [r6 + col], val)
            for col, val in ((0, ua), (1, wa), (2, ta), (3, ub), (4, wb),
                             (5, tb)):
                plsc.store_scatter(dl_v, [r6 + col], val)
            return 0

        lax.fori_loop(0, VITERS, vec, 0)

        sl = pl.ds(base, C)
        pltpu.sync_copy(ul_v, ul_o.at[sl])
        pltpu.sync_copy(phi_v, phi_o.at[sl])
        pltpu.sync_copy(l0_v, l0_o.at[sl])
        pltpu.sync_copy(cc_v, c_o.at[sl])
        pltpu.sync_copy(ss_v, s_o.at[sl])
        pltpu.sync_copy(fl_v, fl_o.at[pl.ds(base * 6, C * 6)])
        pltpu.sync_copy(fg_v, fg_o.at[pl.ds(base * 6, C * 6)])
        pltpu.sync_copy(dl_v, dl_o.at[pl.ds(base * 6, C * 6)])

        def ssub(t, _):
            ssl = pl.ds(t * SUB, SUB)
            pltpu.sync_copy(fax_v.at[ssl], accx.at[idxa2_v.at[t]], add=True)
            pltpu.sync_copy(fay_v.at[ssl], accy.at[idxa2_v.at[t]], add=True)
            pltpu.sync_copy(faz_v.at[ssl], accz.at[idxa2_v.at[t]], add=True)
            pltpu.sync_copy(fbx_v.at[ssl], accx.at[idxb2_v.at[t]], add=True)
            pltpu.sync_copy(fby_v.at[ssl], accy.at[idxb2_v.at[t]], add=True)
            pltpu.sync_copy(fbz_v.at[ssl], accz.at[idxb2_v.at[t]], add=True)
            return 0

        lax.fori_loop(0, NSUB, ssub, 0)
        return 0

    lax.fori_loop(0, cpw, chunk_body, 0)
    plsc.subcore_barrier()

    @pl.when(jnp.logical_and(sub == 0, core == 0))
    def _out0():
        pltpu.sync_copy(accx, p0x_o)
        pltpu.sync_copy(accy, p0y_o)
        pltpu.sync_copy(accz, p0z_o)

    @pl.when(jnp.logical_and(sub == 0, core == 1))
    def _out1():
        pltpu.sync_copy(accx, p1x_o)
        pltpu.sync_copy(accy, p1y_o)
        pltpu.sync_copy(accz, p1z_o)


def kernel(pred_disp, coords, prop_E, prop_A, prop_I22, connectivity):
    n_nodes = pred_disp.shape[0]
    n_edges = connectivity.shape[0]
    assert n_edges % (NW * C) == 0

    f32 = jnp.float32
    tdx = pred_disp[:, 0].astype(f32)
    tdy = pred_disp[:, 1].astype(f32)
    tth = pred_disp[:, 2].astype(f32)
    tcx = coords[:, 0].astype(f32)
    tcz = coords[:, 2].astype(f32)
    na1 = connectivity[:, 0].astype(jnp.int32)
    nb1 = connectivity[:, 1].astype(jnp.int32)
    na3 = na1.reshape(n_edges // C, NSUB, SUB)
    nb3 = nb1.reshape(n_edges // C, NSUB, SUB)
    z1 = jnp.zeros((n_nodes,), f32)

    e1 = jax.ShapeDtypeStruct((n_edges,), f32)
    e6 = jax.ShapeDtypeStruct((n_edges * 6,), f32)
    n1 = jax.ShapeDtypeStruct((n_nodes,), f32)
    out_type = (e6, e6, e6, e1, e1, e1, e1, e1, n1, n1, n1, n1, n1, n1)

    scratch = (
        [pltpu.VMEM((C,), jnp.int32)] * 2 +
        [pltpu.VMEM((NSUB, SUB), jnp.int32)] * 2 +
        [pltpu.VMEM((C,), f32)] * 3 +
        [pltpu.VMEM((C,), f32)] * 10 +
        [pltpu.VMEM((C,), f32)] * 5 +
        [pltpu.VMEM((C * 6,), f32)] * 3 +
        [pltpu.VMEM((C,), f32)] * 6 +
        [pltpu.VMEM_SHARED((n_nodes,), f32)] * 3 +
        [pltpu.SemaphoreType.DMA]
    )

    mesh = plsc.VectorSubcoreMesh(core_axis_name="c", subcore_axis_name="s",
                                  num_cores=NC, num_subcores=NS)
    run = pl.kernel(_beam_body, out_type=out_type, mesh=mesh,
                    scratch_types=scratch,
                    compiler_params=pltpu.CompilerParams(
                        needs_layout_passes=False))
    (fl, fg, dl, u_l, phi, l0, cc, ss,
     p0x, p0y, p0z, p1x, p1y, p1z) = run(
         tdx, tdy, tth, tcx, tcz, na1, nb1, na3, nb3,
         prop_E.astype(f32), prop_A.astype(f32), prop_I22.astype(f32), z1)

    f_local = fl.reshape(n_edges, 6)
    f_global = fg.reshape(n_edges, 6)
    d_local = dl.reshape(n_edges, 6)
    nodal_forces = jnp.stack([p0x + p1x, p0y + p1y, p0z + p1z], axis=1)
    return (f_local[:, 3], f_local[:, 2], f_local[:, 5], f_local[:, 4],
            f_global[:, 0:3], f_global[:, 3:6], nodal_forces,
            f_local, f_global, d_local, u_l, d_local[:, 2], d_local[:, 5],
            phi, l0, cc, ss)


# speedup vs baseline: 21.0574x; 1.6820x over previous
import jax
import jax.numpy as jnp
from jax import lax
from jax.experimental import pallas as pl
from jax.experimental.pallas import tpu as pltpu
from jax.experimental.pallas import tpu_sc as plsc

NC = 2
NS = 16
NW = NC * NS
LANES = 16

C = 2000
SUB = 80
NSUB = C // SUB
VITERS = C // LANES

_MAGIC = 0x5F3759DF


def _rsqrt(x):
    xi = lax.bitcast_convert_type(x, jnp.int32)
    y = lax.bitcast_convert_type(
        jnp.int32(_MAGIC) - lax.shift_right_logical(xi, 1), jnp.float32)
    for _ in range(3):
        y = y * (1.5 - 0.5 * x * y * y)
    return y


def _beam_body_impl(tdx, tdy, tth, tcx, tcz, na1, nb1, na3, nb3,
                    p_e, p_a, p_i, z1,
                    fl_o, fg_o, dl_o, ul_o, phi_o, l0_o, c_o, s_o,
                    p0x_o, p0y_o, p0z_o, p1x_o, p1y_o, p1z_o,
                    idxa_v, idxb_v, idxa2_v, idxb2_v, pe_v, pav_v, pi_v,
                    dax_v, day_v, taa_v, cax_v, caz_v,
                    dbx_v, dby_v, tbb_v, cbx_v, cbz_v,
                    f0_v, f1_v, f2_v, f3_v, f4_v, f5_v,
                    g0_v, g1_v, g3_v, g4_v,
                    ua_v, wa_v, ub_v, wb_v,
                    ul_v, phi_v, l0_v, cc_v, ss_v,
                    accx, accy, accz, sem):
    core = lax.axis_index("c")
    sub = lax.axis_index("s")
    wid = sub * NC + core
    n_edges = ul_o.shape[0]
    cpw = n_edges // (NW * C)

    @pl.when(sub == 0)
    def _zero():
        pltpu.sync_copy(z1, accx)
        pltpu.sync_copy(z1, accy)
        pltpu.sync_copy(z1, accz)

    plsc.subcore_barrier()

    def chunk_body(j, _):
        ck = wid * cpw + j
        base = ck * C
        pltpu.sync_copy(na1.at[pl.ds(base, C)], idxa_v)
        pltpu.sync_copy(nb1.at[pl.ds(base, C)], idxb_v)
        pltpu.sync_copy(na3.at[ck], idxa2_v)
        pltpu.sync_copy(nb3.at[ck], idxb2_v)
        pltpu.sync_copy(p_e.at[pl.ds(base, C)], pe_v)
        pltpu.sync_copy(p_a.at[pl.ds(base, C)], pav_v)
        pltpu.sync_copy(p_i.at[pl.ds(base, C)], pi_v)

        gathers = [
            pltpu.async_copy(tdx.at[idxa_v], dax_v, sem),
            pltpu.async_copy(tdy.at[idxa_v], day_v, sem),
            pltpu.async_copy(tth.at[idxa_v], taa_v, sem),
            pltpu.async_copy(tcx.at[idxa_v], cax_v, sem),
            pltpu.async_copy(tcz.at[idxa_v], caz_v, sem),
            pltpu.async_copy(tdx.at[idxb_v], dbx_v, sem),
            pltpu.async_copy(tdy.at[idxb_v], dby_v, sem),
            pltpu.async_copy(tth.at[idxb_v], tbb_v, sem),
            pltpu.async_copy(tcx.at[idxb_v], cbx_v, sem),
            pltpu.async_copy(tcz.at[idxb_v], cbz_v, sem),
        ]
        for g in gathers:
            g.wait()

        def vec(i, _):
            eb = i * LANES
            sl = pl.ds(eb, LANES)
            d_ax = dax_v[sl]
            d_ay = day_v[sl]
            ta = taa_v[sl]
            cax = cax_v[sl]
            caz = caz_v[sl]
            d_bx = dbx_v[sl]
            d_by = dby_v[sl]
            tb = tbb_v[sl]
            cbx = cbx_v[sl]
            cbz = cbz_v[sl]
            pe = pe_v[sl]
            pa = pav_v[sl]
            pi = pi_v[sl]

            dx0 = cbx - cax
            dz0 = cbz - caz
            x = dx0 * dx0 + dz0 * dz0
            rl = _rsqrt(x)
            l0 = x * rl
            cv = dx0 * rl
            sv = dz0 * rl
            ea = pe * pa
            ei = pe * pi
            ua = cv * d_ax + sv * d_ay
            wa = cv * d_ay - sv * d_ax
            ub = cv * d_bx + sv * d_by
            wb = cv * d_by - sv * d_bx
            rl2 = rl * rl
            rl3 = rl2 * rl
            f0 = ea * rl * (ua - ub)
            wab = wa - wb
            eil3 = ei * rl3
            eil2 = ei * rl2
            eil = ei * rl
            f1 = 12.0 * eil3 * wab + 6.0 * eil2 * (ta + tb)
            f2 = 6.0 * eil2 * wab + eil * (4.0 * ta + 2.0 * tb)
            f5 = 6.0 * eil2 * wab + eil * (2.0 * ta + 4.0 * tb)
            f3 = -f0
            f4 = -f1
            g0 = cv * f0 - sv * f1
            g1 = sv * f0 + cv * f1
            g3 = -g0
            g4 = -g1

            f0_v[sl] = f0
            f1_v[sl] = f1
            f2_v[sl] = f2
            f3_v[sl] = f3
            f4_v[sl] = f4
            f5_v[sl] = f5
            g0_v[sl] = g0
            g1_v[sl] = g1
            g3_v[sl] = g3
            g4_v[sl] = g4
            ua_v[sl] = ua
            wa_v[sl] = wa
            ub_v[sl] = ub
            wb_v[sl] = wb
            ul_v[sl] = ub - ua
            phi_v[sl] = (wb - wa) * rl
            l0_v[sl] = l0
            cc_v[sl] = cv
            ss_v[sl] = sv
            return 0

        lax.fori_loop(0, VITERS, vec, 0)

        for k, buf in enumerate((f0_v, f1_v, f2_v, f3_v, f4_v, f5_v)):
            pltpu.sync_copy(buf, fl_o.at[pl.ds(k * n_edges + base, C)])
        for k, buf in enumerate((g0_v, g1_v, f2_v, g3_v, g4_v, f5_v)):
            pltpu.sync_copy(buf, fg_o.at[pl.ds(k * n_edges + base, C)])
        for k, buf in enumerate((ua_v, wa_v, taa_v, ub_v, wb_v, tbb_v)):
            pltpu.sync_copy(buf, dl_o.at[pl.ds(k * n_edges + base, C)])
        sl = pl.ds(base, C)
        pltpu.sync_copy(ul_v, ul_o.at[sl])
        pltpu.sync_copy(phi_v, phi_o.at[sl])
        pltpu.sync_copy(l0_v, l0_o.at[sl])
        pltpu.sync_copy(cc_v, c_o.at[sl])
        pltpu.sync_copy(ss_v, s_o.at[sl])

        def ssub(t, _):
            ssl = pl.ds(t * SUB, SUB)
            pltpu.sync_copy(g0_v.at[ssl], accx.at[idxa2_v.at[t]], add=True)
            pltpu.sync_copy(g1_v.at[ssl], accy.at[idxa2_v.at[t]], add=True)
            pltpu.sync_copy(f2_v.at[ssl], accz.at[idxa2_v.at[t]], add=True)
            pltpu.sync_copy(g3_v.at[ssl], accx.at[idxb2_v.at[t]], add=True)
            pltpu.sync_copy(g4_v.at[ssl], accy.at[idxb2_v.at[t]], add=True)
            pltpu.sync_copy(f5_v.at[ssl], accz.at[idxb2_v.at[t]], add=True)
            return 0

        lax.fori_loop(0, NSUB, ssub, 0)
        return 0

    lax.fori_loop(0, cpw, chunk_body, 0)
    plsc.subcore_barrier()

    @pl.when(jnp.logical_and(sub == 0, core == 0))
    def _out0():
        pltpu.sync_copy(accx, p0x_o)
        pltpu.sync_copy(accy, p0y_o)
        pltpu.sync_copy(accz, p0z_o)

    @pl.when(jnp.logical_and(sub == 0, core == 1))
    def _out1():
        pltpu.sync_copy(accx, p1x_o)
        pltpu.sync_copy(accy, p1y_o)
        pltpu.sync_copy(accz, p1z_o)


def kernel(pred_disp, coords, prop_E, prop_A, prop_I22, connectivity):
    n_nodes = pred_disp.shape[0]
    n_edges = connectivity.shape[0]
    assert n_edges % (NW * C) == 0

    f32 = jnp.float32
    tdx = pred_disp[:, 0].astype(f32)
    tdy = pred_disp[:, 1].astype(f32)
    tth = pred_disp[:, 2].astype(f32)
    tcx = coords[:, 0].astype(f32)
    tcz = coords[:, 2].astype(f32)
    na1 = connectivity[:, 0].astype(jnp.int32)
    nb1 = connectivity[:, 1].astype(jnp.int32)
    na3 = na1.reshape(n_edges // C, NSUB, SUB)
    nb3 = nb1.reshape(n_edges // C, NSUB, SUB)
    z1 = jnp.zeros((n_nodes,), f32)

    e1 = jax.ShapeDtypeStruct((n_edges,), f32)
    e6 = jax.ShapeDtypeStruct((n_edges * 6,), f32)
    n1 = jax.ShapeDtypeStruct((n_nodes,), f32)
    out_type = (e6, e6, e6, e1, e1, e1, e1, e1, n1, n1, n1, n1, n1, n1)

    scratch = (
        [pltpu.VMEM((C,), jnp.int32)] * 2 +
        [pltpu.VMEM((NSUB, SUB), jnp.int32)] * 2 +
        [pltpu.VMEM((C,), f32)] * 3 +
        [pltpu.VMEM((C,), f32)] * 10 +
        [pltpu.VMEM((C,), f32)] * 6 +
        [pltpu.VMEM((C,), f32)] * 4 +
        [pltpu.VMEM((C,), f32)] * 4 +
        [pltpu.VMEM((C,), f32)] * 5 +
        [pltpu.VMEM_SHARED((n_nodes,), f32)] * 3 +
        [pltpu.SemaphoreType.DMA]
    )

    mesh = plsc.VectorSubcoreMesh(core_axis_name="c", subcore_axis_name="s",
                                  num_cores=NC, num_subcores=NS)
    run = pl.kernel(_beam_body_impl, out_type=out_type, mesh=mesh,
                    scratch_types=scratch,
                    compiler_params=pltpu.CompilerParams(
                        needs_layout_passes=False))
    (fl, fg, dl, u_l, phi, l0, cc, ss,
     p0x, p0y, p0z, p1x, p1y, p1z) = run(
         tdx, tdy, tth, tcx, tcz, na1, nb1, na3, nb3,
         prop_E.astype(f32), prop_A.astype(f32), prop_I22.astype(f32), z1)

    fl6 = fl.reshape(6, n_edges)
    fg6 = fg.reshape(6, n_edges)
    dl6 = dl.reshape(6, n_edges)
    f_local = fl6.T
    f_global = fg6.T
    d_local = dl6.T
    f_ga = fg6[0:3].T
    f_gb = fg6[3:6].T
    nodal_forces = jnp.stack([p0x + p1x, p0y + p1y, p0z + p1z], axis=1)
    return (fl6[3], fl6[2], fl6[5], fl6[4], f_ga, f_gb, nodal_forces,
            f_local, f_global, d_local, u_l, dl6[2], dl6[5],
            phi, l0, cc, ss)


# speedup vs baseline: 62.9519x; 2.9895x over previous
import jax
import jax.numpy as jnp
from jax import lax
from jax.experimental import pallas as pl
from jax.experimental.pallas import tpu as pltpu
from jax.experimental.pallas import tpu_sc as plsc

NC = 2
NS = 16
NW = NC * NS
LANES = 16

C = 2000
VITERS = C // LANES

_MAGIC = 0x5F3759DF


def _rsqrt(x):
    xi = lax.bitcast_convert_type(x, jnp.int32)
    y = lax.bitcast_convert_type(
        jnp.int32(_MAGIC) - lax.shift_right_logical(xi, 1), jnp.float32)
    for _ in range(3):
        y = y * (1.5 - 0.5 * x * y * y)
    return y


def _beam_body(*refs):
    (tdx, tdy, tth, tcx, tcz, na1, nb1, p_e, p_a, p_i, z1) = refs[:11]
    outs_o = refs[11:32]
    (p0x_o, p0y_o, p0z_o, p1x_o, p1y_o, p1z_o) = refs[32:38]
    scr = refs[38:]
    ins = [list(scr[s * 15:(s + 1) * 15]) for s in range(2)]
    outs = list(scr[30:49])
    accx, accy, accz = scr[49:52]
    sem_lin, sem_gat, sem_out, sem_sc = scr[52:56]

    core = lax.axis_index("c")
    sub = lax.axis_index("s")
    wid = sub * NC + core
    n_edges = outs_o[0].shape[0]
    cpw = n_edges // (NW * C)
    assert cpw % 2 == 1 and cpw >= 3

    tables = (tdx, tdy, tth, tcx, tcz)

    def lin_pairs(ck, s):
        sl = pl.ds((wid * cpw + ck) * C, C)
        bufs = ins[s]
        return [(na1.at[sl], bufs[0]), (nb1.at[sl], bufs[1]),
                (p_e.at[sl], bufs[2]), (p_a.at[sl], bufs[3]),
                (p_i.at[sl], bufs[4])]

    def gat_pairs(s):
        bufs = ins[s]
        prs = [(tab.at[bufs[0]], bufs[5 + t]) for t, tab in enumerate(tables)]
        prs += [(tab.at[bufs[1]], bufs[10 + t]) for t, tab in enumerate(tables)]
        return prs

    def out_pairs(ck, s):
        sl = pl.ds((wid * cpw + ck) * C, C)
        prs = [(buf, ref.at[sl]) for buf, ref in zip(outs, (
            outs_o[0], outs_o[1], outs_o[2], outs_o[3], outs_o[4], outs_o[5],
            outs_o[6], outs_o[7], outs_o[8], outs_o[9],
            outs_o[10], outs_o[11], outs_o[13], outs_o[14],
            outs_o[16], outs_o[17], outs_o[18], outs_o[19], outs_o[20]))]
        prs += [(ins[s][7], outs_o[12].at[sl]),
                (ins[s][12], outs_o[15].at[sl])]
        return prs

    def sc_pairs(s):
        ia, ib = ins[s][0], ins[s][1]
        return [(outs[6], accx.at[ia]), (outs[7], accy.at[ia]),
                (outs[2], accz.at[ia]), (outs[8], accx.at[ib]),
                (outs[9], accy.at[ib]), (outs[5], accz.at[ib])]

    def issue(pairs, sem, add=False):
        for src, dst in pairs:
            pltpu.async_copy(src, dst, sem, add=add)

    def drain(pairs, sem):
        for src, dst in pairs:
            pltpu.make_async_copy(src, dst, sem).wait()

    def compute(s):
        bufs = ins[s]
        (f0_v, f1_v, f2_v, f3_v, f4_v, f5_v, g0_v, g1_v, g3_v, g4_v,
         ua_v, wa_v, ub_v, wb_v, ul_v, phi_v, l0_v, cc_v, ss_v) = outs

        def vec(i, _):
            sl = pl.ds(i * LANES, LANES)
            d_ax = bufs[5][sl]
            d_ay = bufs[6][sl]
            ta = bufs[7][sl]
            cax = bufs[8][sl]
            caz = bufs[9][sl]
            d_bx = bufs[10][sl]
            d_by = bufs[11][sl]
            tb = bufs[12][sl]
            cbx = bufs[13][sl]
            cbz = bufs[14][sl]
            pe = bufs[2][sl]
            pa = bufs[3][sl]
            pi = bufs[4][sl]

            dx0 = cbx - cax
            dz0 = cbz - caz
            x = dx0 * dx0 + dz0 * dz0
            rl = _rsqrt(x)
            l0 = x * rl
            cv = dx0 * rl
            sv = dz0 * rl
            ea = pe * pa
            ei = pe * pi
            ua = cv * d_ax + sv * d_ay
            wa = cv * d_ay - sv * d_ax
            ub = cv * d_bx + sv * d_by
            wb = cv * d_by - sv * d_bx
            rl2 = rl * rl
            rl3 = rl2 * rl
            f0 = ea * rl * (ua - ub)
            wab = wa - wb
            eil3 = ei * rl3
            eil2 = ei * rl2
            eil = ei * rl
            f1 = 12.0 * eil3 * wab + 6.0 * eil2 * (ta + tb)
            f2 = 6.0 * eil2 * wab + eil * (4.0 * ta + 2.0 * tb)
            f5 = 6.0 * eil2 * wab + eil * (2.0 * ta + 4.0 * tb)
            f3 = -f0
            f4 = -f1
            g0 = cv * f0 - sv * f1
            g1 = sv * f0 + cv * f1
            g3 = -g0
            g4 = -g1

            f0_v[sl] = f0
            f1_v[sl] = f1
            f2_v[sl] = f2
            f3_v[sl] = f3
            f4_v[sl] = f4
            f5_v[sl] = f5
            g0_v[sl] = g0
            g1_v[sl] = g1
            g3_v[sl] = g3
            g4_v[sl] = g4
            ua_v[sl] = ua
            wa_v[sl] = wa
            ub_v[sl] = ub
            wb_v[sl] = wb
            ul_v[sl] = ub - ua
            phi_v[sl] = (wb - wa) * rl
            l0_v[sl] = l0
            cc_v[sl] = cv
            ss_v[sl] = sv
            return 0

        lax.fori_loop(0, VITERS, vec, 0)

    @pl.when(sub == 0)
    def _zero():
        pltpu.sync_copy(z1, accx)
        pltpu.sync_copy(z1, accy)
        pltpu.sync_copy(z1, accz)

    plsc.subcore_barrier()

    issue(lin_pairs(0, 0), sem_lin)
    drain(lin_pairs(0, 0), sem_lin)
    issue(gat_pairs(0), sem_gat)
    issue(lin_pairs(1, 1), sem_lin)
    drain(gat_pairs(0), sem_gat)
    compute(0)
    issue(out_pairs(0, 0), sem_out)
    issue(sc_pairs(0), sem_sc, add=True)
    drain(lin_pairs(1, 1), sem_lin)
    issue(gat_pairs(1), sem_gat)

    def body(k, _):
        j1 = 2 * k + 1
        j2 = 2 * k + 2
        drain(out_pairs(j1 - 1, 0), sem_out)
        drain(sc_pairs(0), sem_sc)
        issue(lin_pairs(j2, 0), sem_lin)
        drain(gat_pairs(1), sem_gat)
        compute(1)
        issue(out_pairs(j1, 1), sem_out)
        issue(sc_pairs(1), sem_sc, add=True)
        drain(lin_pairs(j2, 0), sem_lin)
        issue(gat_pairs(0), sem_gat)
        drain(out_pairs(j1, 1), sem_out)
        drain(sc_pairs(1), sem_sc)

        @pl.when(k < (cpw - 3) // 2)
        def _pref():
            issue(lin_pairs(j2 + 1, 1), sem_lin)

        drain(gat_pairs(0), sem_gat)
        compute(0)
        issue(out_pairs(j2, 0), sem_out)
        issue(sc_pairs(0), sem_sc, add=True)

        @pl.when(k < (cpw - 3) // 2)
        def _gat():
            drain(lin_pairs(j2 + 1, 1), sem_lin)
            issue(gat_pairs(1), sem_gat)

        return 0

    lax.fori_loop(0, (cpw - 1) // 2, body, 0)
    drain(out_pairs(cpw - 1, 0), sem_out)
    drain(sc_pairs(0), sem_sc)

    plsc.subcore_barrier()

    @pl.when(jnp.logical_and(sub == 0, core == 0))
    def _out0():
        pltpu.sync_copy(accx, p0x_o)
        pltpu.sync_copy(accy, p0y_o)
        pltpu.sync_copy(accz, p0z_o)

    @pl.when(jnp.logical_and(sub == 0, core == 1))
    def _out1():
        pltpu.sync_copy(accx, p1x_o)
        pltpu.sync_copy(accy, p1y_o)
        pltpu.sync_copy(accz, p1z_o)


def kernel(pred_disp, coords, prop_E, prop_A, prop_I22, connectivity):
    n_nodes = pred_disp.shape[0]
    n_edges = connectivity.shape[0]
    assert n_edges % (NW * C) == 0

    f32 = jnp.float32
    tdx = pred_disp[:, 0].astype(f32)
    tdy = pred_disp[:, 1].astype(f32)
    tth = pred_disp[:, 2].astype(f32)
    tcx = coords[:, 0].astype(f32)
    tcz = coords[:, 2].astype(f32)
    na1 = connectivity[:, 0].astype(jnp.int32)
    nb1 = connectivity[:, 1].astype(jnp.int32)
    z1 = jnp.zeros((n_nodes,), f32)

    e1 = jax.ShapeDtypeStruct((n_edges,), f32)
    n1 = jax.ShapeDtypeStruct((n_nodes,), f32)
    out_type = (e1,) * 21 + (n1,) * 6

    set_bufs = ([pltpu.VMEM((C,), jnp.int32)] * 2 +
                [pltpu.VMEM((C,), f32)] * 3 +
                [pltpu.VMEM((C,), f32)] * 10)
    scratch = (
        set_bufs + set_bufs +
        [pltpu.VMEM((C,), f32)] * 19 +
        [pltpu.VMEM_SHARED((n_nodes,), f32)] * 3 +
        [pltpu.SemaphoreType.DMA] * 4
    )

    mesh = plsc.VectorSubcoreMesh(core_axis_name="c", subcore_axis_name="s",
                                  num_cores=NC, num_subcores=NS)
    run = pl.kernel(_beam_body, out_type=out_type, mesh=mesh,
                    scratch_types=scratch,
                    compiler_params=pltpu.CompilerParams(
                        needs_layout_passes=False))
    (f0, f1, f2, f3, f4, f5, g0, g1, g3, g4,
     ua, wa, ta, ub, wb, tb, u_l, phi, l0, cc, ss,
     p0x, p0y, p0z, p1x, p1y, p1z) = run(
         tdx, tdy, tth, tcx, tcz, na1, nb1,
         prop_E.astype(f32), prop_A.astype(f32), prop_I22.astype(f32), z1)

    f_local = jnp.stack([f0, f1, f2, f3, f4, f5], axis=1)
    f_global = jnp.stack([g0, g1, f2, g3, g4, f5], axis=1)
    d_local = jnp.stack([ua, wa, ta, ub, wb, tb], axis=1)
    f_ga = jnp.stack([g0, g1, f2], axis=1)
    f_gb = jnp.stack([g3, g4, f5], axis=1)
    nodal_forces = jnp.stack([p0x + p1x, p0y + p1y, p0z + p1z], axis=1)
    return (f3, f2, f5, f4, f_ga, f_gb, nodal_forces,
            f_local, f_global, d_local, u_l, ta, tb,
            phi, l0, cc, ss)
